# Initial kernel scaffold; baseline (speedup 1.0000x reference)
import jax
import jax.numpy as jnp
from jax.experimental import pallas as pl


def kernel(x, edge_index, batch, params):
    raise NotImplementedError("write your pallas kernel here")


# speedup vs baseline: 3.6095x; 3.6095x over previous
import functools

import jax
import jax.numpy as jnp
from jax import lax
from jax.experimental import pallas as pl
from jax.experimental.pallas import tpu as pltpu
from jax.experimental.pallas import tpu_sc as plsc

_N = 10000
_E = 160000
_D = 512
_NCHUNK = 4
_CD = _D // _NCHUNK
_NC = 2
_NS = 16
_B = 80
_PER_TILE = _E // _NS
_NBLK = _PER_TILE // _B
_FU = 40
_NFU = _N // _FU

_K1, _K2, _K3 = 7000, 4900, 3430


def _edge_body(kt_h, qt_h, vt_h, src_h, dst_h, out_h,
               dstv, srcv, dstg, srcg, krows, qrows, vrows, msg, zbuf,
               aggsh, sem):
    cid = lax.axis_index("c")
    sid = lax.axis_index("s")

    def _zb(r, _):
        for j in range(_CD // 16):
            zbuf[r, pl.ds(j * 16, 16)] = jnp.zeros((16,), jnp.float32)
        return _
    lax.fori_loop(0, _FU, _zb, None)

    nu = (_NFU + _NS - 1) // _NS

    for cc in range(_NCHUNK // _NC):
        c = cid * (_NCHUNK // _NC) + cc
        row0 = c * _N

        def _zero(u, _):
            unit = u * _NS + sid
            @pl.when(unit < _NFU)
            def _():
                pltpu.sync_copy(
                    zbuf, aggsh.at[pl.ds(pl.multiple_of(unit * _FU, _FU), _FU)])
            return _
        lax.fori_loop(0, nu, _zero, None)
        plsc.subcore_barrier()

        def _block(b, _):
            eb = pl.multiple_of(sid * _PER_TILE + b * _B, _B)
            pltpu.sync_copy(dst_h.at[pl.ds(eb, _B)], dstv)
            pltpu.sync_copy(src_h.at[pl.ds(eb, _B)], srcv)
            def _off(i, _c):
                dstg[pl.ds(i * 16, 16)] = dstv[pl.ds(i * 16, 16)] + row0
                srcg[pl.ds(i * 16, 16)] = srcv[pl.ds(i * 16, 16)] + row0
                return _c
            lax.fori_loop(0, _B // 16, _off, None)
            cp1 = pltpu.async_copy(kt_h.at[dstg], krows, sem)
            cp2 = pltpu.async_copy(qt_h.at[srcg], qrows, sem)
            cp3 = pltpu.async_copy(vt_h.at[srcg], vrows, sem)
            cp1.wait()
            cp2.wait()
            cp3.wait()
            def _cmp(e, _c):
                for j in range(_CD // 16):
                    sl = pl.ds(j * 16, 16)
                    kv = krows[e, sl]
                    qv = qrows[e, sl]
                    vv = vrows[e, sl]
                    msg[e, sl] = vv / (1.0 + jnp.exp(-(kv + qv)))
                return _c
            lax.fori_loop(0, _B, _cmp, None)
            pltpu.sync_copy(msg, aggsh.at[dstv], add=True)
            return _
        lax.fori_loop(0, _NBLK, _block, None)
        plsc.subcore_barrier()

        def _flush(u, _):
            unit = u * _NS + sid
            @pl.when(unit < _NFU)
            def _():
                off = pl.multiple_of(unit * _FU, _FU)
                pltpu.sync_copy(aggsh.at[pl.ds(off, _FU)],
                                out_h.at[pl.ds(row0 + off, _FU)])
            return _
        lax.fori_loop(0, nu, _flush, None)
        plsc.subcore_barrier()


_edge_call = functools.partial(
    pl.kernel,
    out_type=jax.ShapeDtypeStruct((_NCHUNK * _N, _CD), jnp.float32),
    mesh=plsc.VectorSubcoreMesh(core_axis_name="c", subcore_axis_name="s",
                                num_cores=_NC, num_subcores=_NS),
    scratch_types=[
        pltpu.VMEM((_B,), jnp.int32),
        pltpu.VMEM((_B,), jnp.int32),
        pltpu.VMEM((_B,), jnp.int32),
        pltpu.VMEM((_B,), jnp.int32),
        pltpu.VMEM((_B, _CD), jnp.float32),
        pltpu.VMEM((_B, _CD), jnp.float32),
        pltpu.VMEM((_B, _CD), jnp.float32),
        pltpu.VMEM((_B, _CD), jnp.float32),
        pltpu.VMEM((_FU, _CD), jnp.float32),
        pltpu.VMEM_SHARED((_N, _CD), jnp.float32),
        pltpu.SemaphoreType.DMA,
    ],
)(_edge_body)


def _to_chunks(a):
    return a.reshape(_N, _NCHUNK, _CD).transpose(1, 0, 2).reshape(_NCHUNK * _N, _CD)


def _from_chunks(a):
    return a.reshape(_NCHUNK, _N, _CD).transpose(1, 0, 2).reshape(_N, _D)


def kernel(x, edge_index, batch, params):
    src = edge_index[0]
    dst = edge_index[1]
    p = params

    def conv(h, cp):
        k = h @ cp['Wk'] + cp['bk']
        q = h @ cp['Wq'] + cp['bq']
        v = h @ cp['Wv'] + cp['bv']
        s = h @ cp['Ws'] + cp['bs']
        agg = _from_chunks(_edge_call(_to_chunks(k), _to_chunks(q),
                                      _to_chunks(v), src, dst))
        return jax.nn.relu(agg + s)

    def pool(h, w, nmask, kk):
        score = jnp.tanh((h @ w) / (jnp.linalg.norm(w) + 1e-16))
        masked = jnp.where(nmask > 0, score, -1e9)
        _, idx = lax.top_k(masked, kk)
        nm = jnp.zeros((_N,), h.dtype).at[idx].set(1.0)
        return h * score[:, None] * nm[:, None], nm

    nmask = jnp.ones((_N,), x.dtype)
    h = conv(x, p['c1'])
    h, nmask = pool(h, p['p1'], nmask, _K1)
    x1 = jnp.sum(h, axis=0) / _K1
    h = conv(h, p['c2'])
    h, nmask = pool(h, p['p2'], nmask, _K2)
    x2 = jnp.sum(h, axis=0) / _K2
    h = conv(h, p['c3'])
    h, nmask = pool(h, p['p3'], nmask, _K3)
    x3 = jnp.sum(h, axis=0) / _K3

    z = (x1 + x2 + x3)[None, :]
    z = jax.nn.relu(z @ p['l1W'] + p['l1b'])
    z = jax.nn.relu(z @ p['l2W'] + p['l2b'])
    return z @ p['l3W'] + p['l3b']


# speedup vs baseline: 4.8467x; 1.3428x over previous
import functools

import jax
import jax.numpy as jnp
from jax import lax
from jax.experimental import pallas as pl
from jax.experimental.pallas import tpu as pltpu
from jax.experimental.pallas import tpu_sc as plsc

_N = 10000
_E = 160000
_D = 512
_NCHUNK = 4
_CD = _D // _NCHUNK
_NC = 2
_NS = 16
_B = 16
_PER_TILE = _E // _NS
_NBLK = _PER_TILE // _B
_FU = 40
_NFU = _N // _FU

_K1, _K2, _K3 = 7000, 4900, 3430

_NPAD = 10112
_SROW = _NPAD // 128


def _edge_body(kt_h, qt_h, vt_h, src_h, dst_h, out_h,
               dsta, srca, dstg0, srcg0, dstg1, srcg1, dsts0, dsts1,
               krows0, qrows0, vrows0, krows1, qrows1, vrows1,
               msg0, msg1, zbuf, aggsh, gsem0, gsem1, ssem0, ssem1):
    cid = lax.axis_index("c")
    sid = lax.axis_index("s")
    e0 = sid * _PER_TILE

    gbuf = ((dstg0, srcg0, dsts0, krows0, qrows0, vrows0, msg0, gsem0, ssem0),
            (dstg1, srcg1, dsts1, krows1, qrows1, vrows1, msg1, gsem1, ssem1))

    pltpu.sync_copy(dst_h.at[pl.ds(e0, _PER_TILE)], dsta)
    pltpu.sync_copy(src_h.at[pl.ds(e0, _PER_TILE)], srca)

    def _zb(r, _):
        for j in range(_CD // 16):
            zbuf[r, pl.ds(j * 16, 16)] = jnp.zeros((16,), jnp.float32)
        return _
    lax.fori_loop(0, _FU, _zb, None)

    nu = (_NFU + _NS - 1) // _NS

    for cc in range(_NCHUNK // _NC):
        c = cid * (_NCHUNK // _NC) + cc
        row0 = c * _N

        def _zero(u, _):
            unit = u * _NS + sid
            @pl.when(unit < _NFU)
            def _():
                pltpu.sync_copy(
                    zbuf, aggsh.at[pl.ds(pl.multiple_of(unit * _FU, _FU), _FU)])
            return _
        lax.fori_loop(0, nu, _zero, None)
        plsc.subcore_barrier()

        def fire(b, slot):
            dstg, srcg, dsts, krows, qrows, vrows, msg, gsem, ssem = gbuf[slot]
            bsl = pl.ds(b * _B, _B)
            dstg[...] = dsta[bsl] + row0
            srcg[...] = srca[bsl] + row0
            pltpu.async_copy(kt_h.at[dstg], krows, gsem)
            pltpu.async_copy(qt_h.at[srcg], qrows, gsem)
            pltpu.async_copy(vt_h.at[srcg], vrows, gsem)

        def finish(b, slot, drain):
            dstg, srcg, dsts, krows, qrows, vrows, msg, gsem, ssem = gbuf[slot]
            pltpu.make_async_copy(kt_h.at[dstg], krows, gsem).wait()
            pltpu.make_async_copy(qt_h.at[srcg], qrows, gsem).wait()
            pltpu.make_async_copy(vt_h.at[srcg], vrows, gsem).wait()
            if drain is None:
                pltpu.make_async_copy(msg, aggsh.at[dsts], ssem).wait()
            elif drain is not False:
                @pl.when(drain)
                def _():
                    pltpu.make_async_copy(msg, aggsh.at[dsts], ssem).wait()
            dsts[...] = dsta[pl.ds(b * _B, _B)]
            def _cmp(g, _c):
                for u in range(4):
                    for j in range(_CD // 16):
                        sl = pl.ds(j * 16, 16)
                        e = g * 4 + u
                        kv = krows[e, sl]
                        qv = qrows[e, sl]
                        vv = vrows[e, sl]
                        msg[e, sl] = vv / (1.0 + jnp.exp(-(kv + qv)))
                return _c
            lax.fori_loop(0, _B // 4, _cmp, None)
            pltpu.async_copy(msg, aggsh.at[dsts], ssem, add=True)

        fire(0, 0)
        def _pair(p, _):
            fire(2 * p + 1, 1)
            finish(2 * p, 0, drain=p > 0)
            fire(2 * p + 2, 0)
            finish(2 * p + 1, 1, drain=p > 0)
            return _
        lax.fori_loop(0, _NBLK // 2, _pair, None)
        finish(_NBLK - 1, 0, drain=None)
        pltpu.make_async_copy(msg0, aggsh.at[dsts0], ssem0).wait()
        pltpu.make_async_copy(msg1, aggsh.at[dsts1], ssem1).wait()
        plsc.subcore_barrier()

        def _flush(u, _):
            unit = u * _NS + sid
            @pl.when(unit < _NFU)
            def _():
                off = pl.multiple_of(unit * _FU, _FU)
                pltpu.sync_copy(aggsh.at[pl.ds(off, _FU)],
                                out_h.at[pl.ds(row0 + off, _FU)])
            return _
        lax.fori_loop(0, nu, _flush, None)
        plsc.subcore_barrier()


def _edge_call(*args):
    return _edge_kernel()(*args)


@functools.cache
def _edge_kernel():
    return functools.partial(
        pl.kernel,
        out_type=jax.ShapeDtypeStruct((_NCHUNK * _N, _CD), jnp.float32),
        mesh=plsc.VectorSubcoreMesh(core_axis_name="c", subcore_axis_name="s",
                                    num_cores=_NC, num_subcores=_NS),
        scratch_types=[
        pltpu.VMEM((_PER_TILE,), jnp.int32),
        pltpu.VMEM((_PER_TILE,), jnp.int32),
        pltpu.VMEM((_B,), jnp.int32),
        pltpu.VMEM((_B,), jnp.int32),
        pltpu.VMEM((_B,), jnp.int32),
        pltpu.VMEM((_B,), jnp.int32),
        pltpu.VMEM((_B,), jnp.int32),
        pltpu.VMEM((_B,), jnp.int32),
        pltpu.VMEM((_B, _CD), jnp.float32),
        pltpu.VMEM((_B, _CD), jnp.float32),
        pltpu.VMEM((_B, _CD), jnp.float32),
        pltpu.VMEM((_B, _CD), jnp.float32),
        pltpu.VMEM((_B, _CD), jnp.float32),
        pltpu.VMEM((_B, _CD), jnp.float32),
        pltpu.VMEM((_B, _CD), jnp.float32),
        pltpu.VMEM((_B, _CD), jnp.float32),
        pltpu.VMEM((_FU, _CD), jnp.float32),
        pltpu.VMEM_SHARED((_N, _CD), jnp.float32),
        pltpu.SemaphoreType.DMA,
        pltpu.SemaphoreType.DMA,
        pltpu.SemaphoreType.DMA,
        pltpu.SemaphoreType.DMA,
        ],
    )(_edge_body)


_RP = 400
_RC = 1000


def _proj_body(x_ref, w_ref, b_ref, o_ref):
    acc = b_ref[...] + jnp.zeros((x_ref.shape[1], 2048), jnp.float32)
    for c in range(x_ref.shape[0]):
        acc = acc + jnp.dot(x_ref[c], w_ref[c],
                            preferred_element_type=jnp.float32)
    for j in range(16):
        o_ref[j] = acc[:, 128 * j:128 * (j + 1)]


def _proj_call(x_cm, w_cm, b_row):
    ic = x_cm.shape[0]
    nt = _N // _RP
    return pl.pallas_call(
        _proj_body,
        grid=(nt,),
        in_specs=[
            pl.BlockSpec((ic, _RP, _CD), lambda i: (0, i, 0)),
            pl.BlockSpec((ic, _CD, 2048), lambda i: (0, 0, 0)),
            pl.BlockSpec((1, 2048), lambda i: (0, 0)),
        ],
        out_specs=pl.BlockSpec((16, _RP, _CD), lambda i: (0, i, 0)),
        out_shape=jax.ShapeDtypeStruct((16, _N, _CD), jnp.float32),
    )(x_cm, w_cm, b_row)


def _combine_body(agg_ref, s_ref, w_ref, h_ref, sc_ref):
    nrm = jnp.sqrt(jnp.sum(w_ref[...] * w_ref[...])) + 1e-16
    acc = jnp.zeros((agg_ref.shape[1], 1), jnp.float32)
    for c in range(_NCHUNK):
        h = jnp.maximum(agg_ref[c] + s_ref[c], 0.0)
        h_ref[c] = h
        acc = acc + jnp.dot(h, w_ref[c][:, None],
                            preferred_element_type=jnp.float32)
    sc_ref[...] = jnp.tanh(acc / nrm)


def _combine_call(agg_cm, s_cm, w_ck):
    nt = _N // _RC
    return pl.pallas_call(
        _combine_body,
        grid=(nt,),
        in_specs=[
            pl.BlockSpec((_NCHUNK, _RC, _CD), lambda i: (0, i, 0)),
            pl.BlockSpec((_NCHUNK, _RC, _CD), lambda i: (0, i, 0)),
            pl.BlockSpec((_NCHUNK, _CD), lambda i: (0, 0)),
        ],
        out_specs=[
            pl.BlockSpec((_NCHUNK, _RC, _CD), lambda i: (0, i, 0)),
            pl.BlockSpec((_RC, 1), lambda i: (i, 0)),
        ],
        out_shape=[
            jax.ShapeDtypeStruct((_NCHUNK, _N, _CD), jnp.float32),
            jax.ShapeDtypeStruct((_N, 1), jnp.float32),
        ],
    )(agg_cm, s_cm, w_ck)


def _sortable(x):
    bits = lax.bitcast_convert_type(x, jnp.int32)
    return jnp.where(bits >= 0, bits, bits ^ jnp.int32(0x7FFFFFFF))


def _topk_body(kk, sc_ref, nm_ref, o_ref):
    masked = jnp.where(nm_ref[...] > 0, sc_ref[...], -1e9)
    key = _sortable(masked)

    def cnt_ge(t):
        return jnp.sum((key >= t).astype(jnp.int32))

    t = jnp.int32(-2147483648)
    for bit in range(31, -1, -1):
        step = jnp.int32(-2147483648) if bit == 31 else jnp.int32(1 << bit)
        cand = t + step
        t = jnp.where(cnt_ge(cand) >= kk, cand, t)

    need = kk - cnt_ge(t + 1)
    idx = (lax.broadcasted_iota(jnp.int32, (_SROW, 128), 0) * 128
           + lax.broadcasted_iota(jnp.int32, (_SROW, 128), 1))
    is_t = key == t
    c = jnp.int32(0)
    for bit in range(13, -1, -1):
        cand = c + jnp.int32(1 << bit)
        tie_cnt = jnp.sum((is_t & (idx < cand)).astype(jnp.int32))
        c = jnp.where(tie_cnt < need, cand, c)
    cstar = c + 1

    lane = lax.broadcasted_iota(jnp.int32, (8, 128), 1)
    o_ref[...] = jnp.where(lane == 1, cstar, t)


def _topk_call(kk, sc2d, nm2d):
    return pl.pallas_call(
        functools.partial(_topk_body, kk),
        grid=(1,),
        in_specs=[
            pl.BlockSpec((_SROW, 128), lambda i: (0, 0)),
            pl.BlockSpec((_SROW, 128), lambda i: (0, 0)),
        ],
        out_specs=pl.BlockSpec((8, 128), lambda i: (0, 0)),
        out_shape=jax.ShapeDtypeStruct((8, 128), jnp.int32),
    )(sc2d, nm2d)


def _gate_body(kk, h_ref, sc_ref, nm_ref, t_ref, hg_ref, nmo_ref, ro_ref):
    i = pl.program_id(0)
    t = t_ref[0, 0]
    cstar = t_ref[0, 1]
    masked = jnp.where(nm_ref[...] > 0, sc_ref[...], -1e9)
    key = _sortable(masked)
    idx = i * _RC + lax.broadcasted_iota(jnp.int32, (_RC, 1), 0)
    sel = (key > t) | ((key == t) & (idx < cstar))
    keep = sel.astype(jnp.float32)
    nmo_ref[...] = keep
    gate = sc_ref[...] * keep

    @pl.when(i == 0)
    def _():
        ro_ref[...] = jnp.zeros((_NCHUNK, _CD), jnp.float32)

    parts = []
    for c in range(_NCHUNK):
        hg = h_ref[c] * gate
        hg_ref[c] = hg
        parts.append(jnp.sum(hg, axis=0, keepdims=True) * (1.0 / kk))
    ro_ref[...] = ro_ref[...] + jnp.concatenate(parts, axis=0)


def _gate_call(kk, h_cm, sc_col, nm_col, taux):
    nt = _N // _RC
    return pl.pallas_call(
        functools.partial(_gate_body, kk),
        grid=(nt,),
        in_specs=[
            pl.BlockSpec((_NCHUNK, _RC, _CD), lambda i: (0, i, 0)),
            pl.BlockSpec((_RC, 1), lambda i: (i, 0)),
            pl.BlockSpec((_RC, 1), lambda i: (i, 0)),
            pl.BlockSpec((8, 128), lambda i: (0, 0)),
        ],
        out_specs=[
            pl.BlockSpec((_NCHUNK, _RC, _CD), lambda i: (0, i, 0)),
            pl.BlockSpec((_RC, 1), lambda i: (i, 0)),
            pl.BlockSpec((_NCHUNK, _CD), lambda i: (0, 0)),
        ],
        out_shape=[
            jax.ShapeDtypeStruct((_NCHUNK, _N, _CD), jnp.float32),
            jax.ShapeDtypeStruct((_N, 1), jnp.float32),
            jax.ShapeDtypeStruct((_NCHUNK, _CD), jnp.float32),
        ],
    )(h_cm, sc_col, nm_col, taux)


def _mlp_body(r1_ref, r2_ref, r3_ref, w1_ref, b1_ref, w2_ref, b2_ref,
              w3_ref, b3_ref, o_ref):
    z = r1_ref[...] + r2_ref[...] + r3_ref[...]
    z = jnp.maximum(jnp.dot(z, w1_ref[...],
                            preferred_element_type=jnp.float32)
                    + b1_ref[...], 0.0)
    z = jnp.maximum(jnp.dot(z, w2_ref[...],
                            preferred_element_type=jnp.float32)
                    + b2_ref[...], 0.0)
    o_ref[...] = jnp.dot(z, w3_ref[...],
                         preferred_element_type=jnp.float32) + b3_ref[...]


def _mlp_call(r1, r2, r3, w1, b1, w2, b2, w3, b3):
    no = w3.shape[1]
    specs = [pl.BlockSpec(a.shape, lambda i: tuple(0 for _ in a.shape))
             for a in (r1, r2, r3, w1, b1, w2, b2, w3, b3)]
    return pl.pallas_call(
        _mlp_body,
        grid=(1,),
        in_specs=specs,
        out_specs=pl.BlockSpec((1, no), lambda i: (0, 0)),
        out_shape=jax.ShapeDtypeStruct((1, no), jnp.float32),
    )(r1, r2, r3, w1, b1, w2, b2, w3, b3)


def _pad_img(col, fill):
    return jnp.pad(col[:, 0], (0, _NPAD - _N),
                   constant_values=fill).reshape(_SROW, 128)


def kernel(x, edge_index, batch, params):
    src = edge_index[0]
    dst = edge_index[1]
    p = params

    def layer(h_cm, cp, w_pool, kk, nm_col):
        ic = h_cm.shape[0]
        w_cm = jnp.concatenate([cp['Wk'], cp['Wq'], cp['Wv'], cp['Ws']],
                               axis=1).reshape(ic, _CD, 2048)
        b_row = jnp.concatenate([cp['bk'], cp['bq'], cp['bv'], cp['bs']])[None]
        y = _proj_call(h_cm, w_cm, b_row)
        k = y[0:4].reshape(_NCHUNK * _N, _CD)
        q = y[4:8].reshape(_NCHUNK * _N, _CD)
        v = y[8:12].reshape(_NCHUNK * _N, _CD)
        s = y[12:16]
        agg = _edge_call(k, q, v, src, dst).reshape(_NCHUNK, _N, _CD)
        h4, score = _combine_call(agg, s, w_pool.reshape(_NCHUNK, _CD))
        taux = _topk_call(kk, _pad_img(score, 0.0), _pad_img(nm_col, 0.0))
        hg, nm_new, ro = _gate_call(kk, h4, score, nm_col, taux)
        return hg, nm_new, ro

    x_cm = x.reshape(_N, 2, _CD).transpose(1, 0, 2)
    nmask = jnp.ones((_N, 1), jnp.float32)
    h, nmask, r1 = layer(x_cm, p['c1'], p['p1'], _K1, nmask)
    h, nmask, r2 = layer(h, p['c2'], p['p2'], _K2, nmask)
    h, nmask, r3 = layer(h, p['c3'], p['p3'], _K3, nmask)

    return _mlp_call(r1.reshape(1, _D), r2.reshape(1, _D), r3.reshape(1, _D),
                     p['l1W'], p['l1b'][None], p['l2W'], p['l2b'][None],
                     p['l3W'], p['l3b'][None])


# speedup vs baseline: 4.8918x; 1.0093x over previous
import functools

import jax
import jax.numpy as jnp
from jax import lax
from jax.experimental import pallas as pl
from jax.experimental.pallas import tpu as pltpu
from jax.experimental.pallas import tpu_sc as plsc

_N = 10000
_E = 160000
_D = 512
_NCHUNK = 4
_CD = _D // _NCHUNK
_NC = 2
_NS = 16
_B = 16
_PER_TILE = _E // _NS
_NBLK = _PER_TILE // _B
_FU = 40
_NFU = _N // _FU

_K1, _K2, _K3 = 7000, 4900, 3430

_NPAD = 10112
_SROW = _NPAD // 128


_DUMMY = 16383 * 16384
_AGG_ROWS = _N + 8


def _edge_body(pk_h, nm_h, kt_h, qt_h, vt_h, out_h,
               pka, dstg0, srcg0, dstg1, srcg1, dsts0, dsts1,
               krows0, qrows0, vrows0, krows1, qrows1, vrows1,
               msg0, msg1, zbuf, aggsh, gsem0, gsem1, ssem0, ssem1):
    cid = lax.axis_index("c")
    sid = lax.axis_index("s")
    e0 = sid * _PER_TILE

    gbuf = ((dstg0, srcg0, dsts0, krows0, qrows0, vrows0, msg0, gsem0, ssem0),
            (dstg1, srcg1, dsts1, krows1, qrows1, vrows1, msg1, gsem1, ssem1))

    pltpu.sync_copy(pk_h.at[pl.ds(e0, _PER_TILE)], pka.at[pl.ds(0, _PER_TILE)])

    nb = _PER_TILE // _B

    def _zb(r, _):
        for j in range(_CD // 16):
            zbuf[r, pl.ds(j * 16, 16)] = jnp.zeros((16,), jnp.float32)
        return _
    lax.fori_loop(0, _FU, _zb, None)

    nu = (_NFU + _NS - 1) // _NS

    for cc in range(_NCHUNK // _NC):
        c = cid * (_NCHUNK // _NC) + cc
        row0 = c * _N

        for u in range(nu):
            unit = u * _NS + sid
            @pl.when(unit < _NFU)
            def _():
                pltpu.async_copy(
                    zbuf, aggsh.at[pl.ds(pl.multiple_of(unit * _FU, _FU), _FU)],
                    gsem0)
        for u in range(nu):
            unit = u * _NS + sid
            @pl.when(unit < _NFU)
            def _():
                pltpu.make_async_copy(
                    zbuf, aggsh.at[pl.ds(pl.multiple_of(unit * _FU, _FU), _FU)],
                    gsem0).wait()
        plsc.subcore_barrier()

        def fire(b, slot):
            dstg, srcg, dsts, krows, qrows, vrows, msg, gsem, ssem = gbuf[slot]
            grp = pka[pl.ds(b * _B, _B)]
            dv = lax.shift_right_logical(grp, 14)
            dstg[...] = jnp.where(dv > _N - 1, _N - 1, dv) + row0
            srcg[...] = (grp & 16383) + row0
            pltpu.async_copy(kt_h.at[dstg, :], krows, gsem)
            pltpu.async_copy(qt_h.at[srcg, :], qrows, gsem)
            pltpu.async_copy(vt_h.at[srcg, :], vrows, gsem)

        def finish(b, slot):
            dstg, srcg, dsts, krows, qrows, vrows, msg, gsem, ssem = gbuf[slot]
            pltpu.make_async_copy(kt_h.at[dstg, :], krows, gsem).wait()
            pltpu.make_async_copy(qt_h.at[srcg, :], qrows, gsem).wait()
            pltpu.make_async_copy(vt_h.at[srcg, :], vrows, gsem).wait()
            @pl.when(b >= 2)
            def _():
                pltpu.make_async_copy(msg, aggsh.at[dsts, :], ssem).wait()
            grp = pka[pl.ds(b * _B, _B)]
            dv = lax.shift_right_logical(grp, 14)
            dsts[...] = jnp.where(dv > _N, _N, dv)
            def _cmp(g, _c):
                for u in range(4):
                    for j in range(_CD // 16):
                        sl = pl.ds(j * 16, 16)
                        e = g * 4 + u
                        kv = krows[e, sl]
                        qv = qrows[e, sl]
                        vv = vrows[e, sl]
                        msg[e, sl] = vv / (1.0 + jnp.exp(-(kv + qv)))
                return _c
            lax.fori_loop(0, _B // 4, _cmp, None)
            pltpu.async_copy(msg, aggsh.at[dsts, :], ssem, add=True)

        @pl.when(nb > 0)
        def _():
            fire(0, 0)

        def _blk(b, _):
            nxt = b + 1
            @pl.when((nxt < nb) & (nxt % 2 == 0))
            def _():
                fire(nxt, 0)
            @pl.when((nxt < nb) & (nxt % 2 == 1))
            def _():
                fire(nxt, 1)
            @pl.when(b % 2 == 0)
            def _():
                finish(b, 0)
            @pl.when(b % 2 == 1)
            def _():
                finish(b, 1)
            return _
        lax.fori_loop(0, nb, _blk, None)
        @pl.when(nb >= 1)
        def _():
            pltpu.make_async_copy(msg0, aggsh.at[dsts0, :], ssem0).wait()
        @pl.when(nb >= 2)
        def _():
            pltpu.make_async_copy(msg1, aggsh.at[dsts1, :], ssem1).wait()
        plsc.subcore_barrier()

        for u in range(nu):
            unit = u * _NS + sid
            @pl.when(unit < _NFU)
            def _():
                off = pl.multiple_of(unit * _FU, _FU)
                pltpu.async_copy(aggsh.at[pl.ds(off, _FU)],
                                 out_h.at[pl.ds(row0 + off, _FU)], gsem0)
        for u in range(nu):
            unit = u * _NS + sid
            @pl.when(unit < _NFU)
            def _():
                off = pl.multiple_of(unit * _FU, _FU)
                pltpu.make_async_copy(aggsh.at[pl.ds(off, _FU)],
                                      out_h.at[pl.ds(row0 + off, _FU)],
                                      gsem0).wait()
        plsc.subcore_barrier()


def _edge_call(*args):
    return _edge_kernel()(*args)


@functools.cache
def _edge_kernel():
    return functools.partial(
        pl.kernel,
        out_type=jax.ShapeDtypeStruct((_NCHUNK * _N, _CD), jnp.float32),
        mesh=plsc.VectorSubcoreMesh(core_axis_name="c", subcore_axis_name="s",
                                    num_cores=_NC, num_subcores=_NS),
        scratch_types=[
        pltpu.VMEM((_PER_TILE + _B,), jnp.int32),
        pltpu.VMEM((_B,), jnp.int32),
        pltpu.VMEM((_B,), jnp.int32),
        pltpu.VMEM((_B,), jnp.int32),
        pltpu.VMEM((_B,), jnp.int32),
        pltpu.VMEM((_B,), jnp.int32),
        pltpu.VMEM((_B,), jnp.int32),
        pltpu.VMEM((_B, _CD), jnp.float32),
        pltpu.VMEM((_B, _CD), jnp.float32),
        pltpu.VMEM((_B, _CD), jnp.float32),
        pltpu.VMEM((_B, _CD), jnp.float32),
        pltpu.VMEM((_B, _CD), jnp.float32),
        pltpu.VMEM((_B, _CD), jnp.float32),
        pltpu.VMEM((_B, _CD), jnp.float32),
        pltpu.VMEM((_B, _CD), jnp.float32),
        pltpu.VMEM((_FU, _CD), jnp.float32),
        pltpu.VMEM_SHARED((_AGG_ROWS, _CD), jnp.float32),
        pltpu.SemaphoreType.DMA,
        pltpu.SemaphoreType.DMA,
        pltpu.SemaphoreType.DMA,
        pltpu.SemaphoreType.DMA,
        ],
    )(_edge_body)


_RP = 400
_RC = 1000


def _proj_body(x_ref, w_ref, b_ref, o_ref):
    acc = b_ref[...] + jnp.zeros((x_ref.shape[1], 2048), jnp.float32)
    for c in range(x_ref.shape[0]):
        acc = acc + jnp.dot(x_ref[c], w_ref[c],
                            preferred_element_type=jnp.float32)
    for j in range(16):
        o_ref[j] = acc[:, 128 * j:128 * (j + 1)]


def _proj_call(x_cm, w_cm, b_row):
    ic = x_cm.shape[0]
    nt = _N // _RP
    return pl.pallas_call(
        _proj_body,
        grid=(nt,),
        in_specs=[
            pl.BlockSpec((ic, _RP, _CD), lambda i: (0, i, 0)),
            pl.BlockSpec((ic, _CD, 2048), lambda i: (0, 0, 0)),
            pl.BlockSpec((1, 2048), lambda i: (0, 0)),
        ],
        out_specs=pl.BlockSpec((16, _RP, _CD), lambda i: (0, i, 0)),
        out_shape=jax.ShapeDtypeStruct((16, _N, _CD), jnp.float32),
    )(x_cm, w_cm, b_row)


def _combine_body(agg_ref, s_ref, w_ref, h_ref, sc_ref):
    nrm = jnp.sqrt(jnp.sum(w_ref[...] * w_ref[...])) + 1e-16
    acc = jnp.zeros((agg_ref.shape[1], 1), jnp.float32)
    for c in range(_NCHUNK):
        h = jnp.maximum(agg_ref[c] + s_ref[c], 0.0)
        h_ref[c] = h
        acc = acc + jnp.dot(h, w_ref[c][:, None],
                            preferred_element_type=jnp.float32)
    sc_ref[...] = jnp.tanh(acc / nrm)


def _combine_call(agg_cm, s_cm, w_ck):
    nt = _N // _RC
    return pl.pallas_call(
        _combine_body,
        grid=(nt,),
        in_specs=[
            pl.BlockSpec((_NCHUNK, _RC, _CD), lambda i: (0, i, 0)),
            pl.BlockSpec((_NCHUNK, _RC, _CD), lambda i: (0, i, 0)),
            pl.BlockSpec((_NCHUNK, _CD), lambda i: (0, 0)),
        ],
        out_specs=[
            pl.BlockSpec((_NCHUNK, _RC, _CD), lambda i: (0, i, 0)),
            pl.BlockSpec((_RC, 1), lambda i: (i, 0)),
        ],
        out_shape=[
            jax.ShapeDtypeStruct((_NCHUNK, _N, _CD), jnp.float32),
            jax.ShapeDtypeStruct((_N, 1), jnp.float32),
        ],
    )(agg_cm, s_cm, w_ck)


def _sortable(x):
    bits = lax.bitcast_convert_type(x, jnp.int32)
    return jnp.where(bits >= 0, bits, bits ^ jnp.int32(0x7FFFFFFF))


def _topk_body(kk, sc_ref, nm_ref, o_ref):
    masked = jnp.where(nm_ref[...] > 0, sc_ref[...], -1e9)
    key = _sortable(masked)

    def cnt_ge(t):
        return jnp.sum((key >= t).astype(jnp.int32))

    t = jnp.int32(-2147483648)
    for bit in range(31, -1, -1):
        step = jnp.int32(-2147483648) if bit == 31 else jnp.int32(1 << bit)
        cand = t + step
        t = jnp.where(cnt_ge(cand) >= kk, cand, t)

    need = kk - cnt_ge(t + 1)
    idx = (lax.broadcasted_iota(jnp.int32, (_SROW, 128), 0) * 128
           + lax.broadcasted_iota(jnp.int32, (_SROW, 128), 1))
    is_t = key == t
    c = jnp.int32(0)
    for bit in range(13, -1, -1):
        cand = c + jnp.int32(1 << bit)
        tie_cnt = jnp.sum((is_t & (idx < cand)).astype(jnp.int32))
        c = jnp.where(tie_cnt < need, cand, c)
    cstar = c + 1

    lane = lax.broadcasted_iota(jnp.int32, (8, 128), 1)
    o_ref[...] = jnp.where(lane == 1, cstar, t)


def _topk_call(kk, sc2d, nm2d):
    return pl.pallas_call(
        functools.partial(_topk_body, kk),
        grid=(1,),
        in_specs=[
            pl.BlockSpec((_SROW, 128), lambda i: (0, 0)),
            pl.BlockSpec((_SROW, 128), lambda i: (0, 0)),
        ],
        out_specs=pl.BlockSpec((8, 128), lambda i: (0, 0)),
        out_shape=jax.ShapeDtypeStruct((8, 128), jnp.int32),
    )(sc2d, nm2d)


def _gate_body(kk, h_ref, sc_ref, nm_ref, t_ref, hg_ref, nmo_ref, ro_ref):
    i = pl.program_id(0)
    t = t_ref[0, 0]
    cstar = t_ref[0, 1]
    masked = jnp.where(nm_ref[...] > 0, sc_ref[...], -1e9)
    key = _sortable(masked)
    idx = i * _RC + lax.broadcasted_iota(jnp.int32, (_RC, 1), 0)
    sel = (key > t) | ((key == t) & (idx < cstar))
    keep = sel.astype(jnp.float32)
    nmo_ref[...] = keep
    gate = sc_ref[...] * keep

    @pl.when(i == 0)
    def _():
        ro_ref[...] = jnp.zeros((_NCHUNK, _CD), jnp.float32)

    parts = []
    for c in range(_NCHUNK):
        hg = h_ref[c] * gate
        hg_ref[c] = hg
        parts.append(jnp.sum(hg, axis=0, keepdims=True) * (1.0 / kk))
    ro_ref[...] = ro_ref[...] + jnp.concatenate(parts, axis=0)


def _gate_call(kk, h_cm, sc_col, nm_col, taux):
    nt = _N // _RC
    return pl.pallas_call(
        functools.partial(_gate_body, kk),
        grid=(nt,),
        in_specs=[
            pl.BlockSpec((_NCHUNK, _RC, _CD), lambda i: (0, i, 0)),
            pl.BlockSpec((_RC, 1), lambda i: (i, 0)),
            pl.BlockSpec((_RC, 1), lambda i: (i, 0)),
            pl.BlockSpec((8, 128), lambda i: (0, 0)),
        ],
        out_specs=[
            pl.BlockSpec((_NCHUNK, _RC, _CD), lambda i: (0, i, 0)),
            pl.BlockSpec((_RC, 1), lambda i: (i, 0)),
            pl.BlockSpec((_NCHUNK, _CD), lambda i: (0, 0)),
        ],
        out_shape=[
            jax.ShapeDtypeStruct((_NCHUNK, _N, _CD), jnp.float32),
            jax.ShapeDtypeStruct((_N, 1), jnp.float32),
            jax.ShapeDtypeStruct((_NCHUNK, _CD), jnp.float32),
        ],
    )(h_cm, sc_col, nm_col, taux)


def _mlp_body(r1_ref, r2_ref, r3_ref, w1_ref, b1_ref, w2_ref, b2_ref,
              w3_ref, b3_ref, o_ref):
    z = r1_ref[...] + r2_ref[...] + r3_ref[...]
    z = jnp.maximum(jnp.dot(z, w1_ref[...],
                            preferred_element_type=jnp.float32)
                    + b1_ref[...], 0.0)
    z = jnp.maximum(jnp.dot(z, w2_ref[...],
                            preferred_element_type=jnp.float32)
                    + b2_ref[...], 0.0)
    o_ref[...] = jnp.dot(z, w3_ref[...],
                         preferred_element_type=jnp.float32) + b3_ref[...]


def _mlp_call(r1, r2, r3, w1, b1, w2, b2, w3, b3):
    no = w3.shape[1]
    specs = [pl.BlockSpec(a.shape, lambda i: tuple(0 for _ in a.shape))
             for a in (r1, r2, r3, w1, b1, w2, b2, w3, b3)]
    return pl.pallas_call(
        _mlp_body,
        grid=(1,),
        in_specs=specs,
        out_specs=pl.BlockSpec((1, no), lambda i: (0, 0)),
        out_shape=jax.ShapeDtypeStruct((1, no), jnp.float32),
    )(r1, r2, r3, w1, b1, w2, b2, w3, b3)


def _pad_img(col, fill):
    return jnp.pad(col[:, 0], (0, _NPAD - _N),
                   constant_values=fill).reshape(_SROW, 128)


def kernel(x, edge_index, batch, params):
    src = edge_index[0].astype(jnp.int32)
    dst = edge_index[1].astype(jnp.int32)
    pk = (dst * 16384 + src).astype(jnp.int32)
    p = params

    def layer(h_cm, cp, w_pool, kk, nm_col):
        ic = h_cm.shape[0]
        w_cm = jnp.concatenate([cp['Wk'], cp['Wq'], cp['Wv'], cp['Ws']],
                               axis=1).reshape(ic, _CD, 2048)
        b_row = jnp.concatenate([cp['bk'], cp['bq'], cp['bv'], cp['bs']])[None]
        y = _proj_call(h_cm, w_cm, b_row)
        k = y[0:4].reshape(_NCHUNK * _N, _CD)
        q = y[4:8].reshape(_NCHUNK * _N, _CD)
        v = y[8:12].reshape(_NCHUNK * _N, _CD)
        s = y[12:16]
        agg = _edge_call(pk, nm_col[:, 0].astype(jnp.int32), k, q, v).reshape(_NCHUNK, _N, _CD)
        h4, score = _combine_call(agg, s, w_pool.reshape(_NCHUNK, _CD))
        taux = _topk_call(kk, _pad_img(score, 0.0), _pad_img(nm_col, 0.0))
        hg, nm_new, ro = _gate_call(kk, h4, score, nm_col, taux)
        return hg, nm_new, ro

    x_cm = x.reshape(_N, 2, _CD).transpose(1, 0, 2)
    nmask = jnp.ones((_N, 1), jnp.float32)
    h, nmask, r1 = layer(x_cm, p['c1'], p['p1'], _K1, nmask)
    h, nmask, r2 = layer(h, p['c2'], p['p2'], _K2, nmask)
    h, nmask, r3 = layer(h, p['c3'], p['p3'], _K3, nmask)

    return _mlp_call(r1.reshape(1, _D), r2.reshape(1, _D), r3.reshape(1, _D),
                     p['l1W'], p['l1b'][None], p['l2W'], p['l2b'][None],
                     p['l3W'], p['l3b'][None])


# speedup vs baseline: 4.9063x; 1.0030x over previous
import functools

import jax
import jax.numpy as jnp
from jax import lax
from jax.experimental import pallas as pl
from jax.experimental.pallas import tpu as pltpu
from jax.experimental.pallas import tpu_sc as plsc

_N = 10000
_E = 160000
_D = 512
_NCHUNK = 4
_CD = _D // _NCHUNK
_NC = 2
_NS = 16
_B = 16
_PER_TILE = _E // _NS
_NBLK = _PER_TILE // _B
_FU = 40
_NFU = _N // _FU

_K1, _K2, _K3 = 7000, 4900, 3430

_NPAD = 10112
_SROW = _NPAD // 128


_AGG_ROWS = _N + 8


def _edge_body(pk_h, kt_h, qt_h, vt_h, out_h,
               pka, dstg0, srcg0, dstg1, srcg1, dsts0, dsts1,
               krows0, qrows0, vrows0, krows1, qrows1, vrows1,
               msg0, msg1, zbuf, aggsh, gsem0, gsem1, ssem0, ssem1):
    cid = lax.axis_index("c")
    sid = lax.axis_index("s")
    e0 = sid * _PER_TILE

    gbuf = ((dstg0, srcg0, dsts0, krows0, qrows0, vrows0, msg0, gsem0, ssem0),
            (dstg1, srcg1, dsts1, krows1, qrows1, vrows1, msg1, gsem1, ssem1))

    pltpu.sync_copy(pk_h.at[pl.ds(e0, _PER_TILE)], pka.at[pl.ds(0, _PER_TILE)])
    nb = _NBLK

    def _zb(r, _):
        for j in range(_CD // 16):
            zbuf[r, pl.ds(j * 16, 16)] = jnp.zeros((16,), jnp.float32)
        return _
    lax.fori_loop(0, _FU, _zb, None)

    nu = (_NFU + _NS - 1) // _NS

    for cc in range(_NCHUNK // _NC):
        c = cid * (_NCHUNK // _NC) + cc
        row0 = c * _N

        for u in range(nu):
            unit = u * _NS + sid
            @pl.when(unit < _NFU)
            def _():
                pltpu.async_copy(
                    zbuf, aggsh.at[pl.ds(pl.multiple_of(unit * _FU, _FU), _FU)],
                    gsem0)
        for u in range(nu):
            unit = u * _NS + sid
            @pl.when(unit < _NFU)
            def _():
                pltpu.make_async_copy(
                    zbuf, aggsh.at[pl.ds(pl.multiple_of(unit * _FU, _FU), _FU)],
                    gsem0).wait()
        plsc.subcore_barrier()

        def fire(b, slot):
            dstg, srcg, dsts, krows, qrows, vrows, msg, gsem, ssem = gbuf[slot]
            grp = pka[pl.ds(b * _B, _B)]
            dstg[...] = lax.shift_right_logical(grp, 14) + row0
            srcg[...] = (grp & 16383) + row0
            pltpu.async_copy(kt_h.at[dstg, :], krows, gsem)
            pltpu.async_copy(qt_h.at[srcg, :], qrows, gsem)
            pltpu.async_copy(vt_h.at[srcg, :], vrows, gsem)

        def finish(b, slot):
            dstg, srcg, dsts, krows, qrows, vrows, msg, gsem, ssem = gbuf[slot]
            pltpu.make_async_copy(kt_h.at[dstg, :], krows, gsem).wait()
            pltpu.make_async_copy(qt_h.at[srcg, :], qrows, gsem).wait()
            pltpu.make_async_copy(vt_h.at[srcg, :], vrows, gsem).wait()
            @pl.when(b >= 2)
            def _():
                pltpu.make_async_copy(msg, aggsh.at[dsts, :], ssem).wait()
            grp = pka[pl.ds(b * _B, _B)]
            dsts[...] = lax.shift_right_logical(grp, 14)
            def _cmp(g, _c):
                for u in range(4):
                    for j in range(_CD // 16):
                        sl = pl.ds(j * 16, 16)
                        e = g * 4 + u
                        kv = krows[e, sl]
                        qv = qrows[e, sl]
                        vv = vrows[e, sl]
                        msg[e, sl] = vv / (1.0 + jnp.exp(-(kv + qv)))
                return _c
            lax.fori_loop(0, _B // 4, _cmp, None)
            pltpu.async_copy(msg, aggsh.at[dsts, :], ssem, add=True)

        @pl.when(nb > 0)
        def _():
            fire(0, 0)

        def _blk(b, _):
            nxt = b + 1
            @pl.when((nxt < nb) & (nxt % 2 == 0))
            def _():
                fire(nxt, 0)
            @pl.when((nxt < nb) & (nxt % 2 == 1))
            def _():
                fire(nxt, 1)
            @pl.when(b % 2 == 0)
            def _():
                finish(b, 0)
            @pl.when(b % 2 == 1)
            def _():
                finish(b, 1)
            return _
        lax.fori_loop(0, nb, _blk, None)
        @pl.when(nb >= 1)
        def _():
            pltpu.make_async_copy(msg0, aggsh.at[dsts0, :], ssem0).wait()
        @pl.when(nb >= 2)
        def _():
            pltpu.make_async_copy(msg1, aggsh.at[dsts1, :], ssem1).wait()
        plsc.subcore_barrier()

        for u in range(nu):
            unit = u * _NS + sid
            @pl.when(unit < _NFU)
            def _():
                off = pl.multiple_of(unit * _FU, _FU)
                pltpu.async_copy(aggsh.at[pl.ds(off, _FU)],
                                 out_h.at[pl.ds(row0 + off, _FU)], gsem0)
        for u in range(nu):
            unit = u * _NS + sid
            @pl.when(unit < _NFU)
            def _():
                off = pl.multiple_of(unit * _FU, _FU)
                pltpu.make_async_copy(aggsh.at[pl.ds(off, _FU)],
                                      out_h.at[pl.ds(row0 + off, _FU)],
                                      gsem0).wait()
        plsc.subcore_barrier()


def _edge_call(*args):
    return _edge_kernel()(*args)


@functools.cache
def _edge_kernel():
    return functools.partial(
        pl.kernel,
        out_type=[jax.ShapeDtypeStruct((_NCHUNK * _N, _CD), jnp.float32)],
        mesh=plsc.VectorSubcoreMesh(core_axis_name="c", subcore_axis_name="s",
                                    num_cores=_NC, num_subcores=_NS),
        scratch_types=[
        pltpu.VMEM((_PER_TILE,), jnp.int32),
        pltpu.VMEM((_B,), jnp.int32),
        pltpu.VMEM((_B,), jnp.int32),
        pltpu.VMEM((_B,), jnp.int32),
        pltpu.VMEM((_B,), jnp.int32),
        pltpu.VMEM((_B,), jnp.int32),
        pltpu.VMEM((_B,), jnp.int32),
        pltpu.VMEM((_B, _CD), jnp.float32),
        pltpu.VMEM((_B, _CD), jnp.float32),
        pltpu.VMEM((_B, _CD), jnp.float32),
        pltpu.VMEM((_B, _CD), jnp.float32),
        pltpu.VMEM((_B, _CD), jnp.float32),
        pltpu.VMEM((_B, _CD), jnp.float32),
        pltpu.VMEM((_B, _CD), jnp.float32),
        pltpu.VMEM((_B, _CD), jnp.float32),
        pltpu.VMEM((_FU, _CD), jnp.float32),
        pltpu.VMEM_SHARED((_AGG_ROWS, _CD), jnp.float32),
        pltpu.SemaphoreType.DMA,
        pltpu.SemaphoreType.DMA,
        pltpu.SemaphoreType.DMA,
        pltpu.SemaphoreType.DMA,
        ],
    )(_edge_body)


_RP = 400
_RC = 1000


def _proj_body(x_ref, w_ref, b_ref, o_ref):
    acc = b_ref[...] + jnp.zeros((x_ref.shape[1], 2048), jnp.float32)
    for c in range(x_ref.shape[0]):
        acc = acc + jnp.dot(x_ref[c], w_ref[c],
                            preferred_element_type=jnp.float32)
    for j in range(16):
        o_ref[j] = acc[:, 128 * j:128 * (j + 1)]


def _proj_call(x_cm, w_cm, b_row):
    ic = x_cm.shape[0]
    nt = _N // _RP
    return pl.pallas_call(
        _proj_body,
        grid=(nt,),
        in_specs=[
            pl.BlockSpec((ic, _RP, _CD), lambda i: (0, i, 0)),
            pl.BlockSpec((ic, _CD, 2048), lambda i: (0, 0, 0)),
            pl.BlockSpec((1, 2048), lambda i: (0, 0)),
        ],
        out_specs=pl.BlockSpec((16, _RP, _CD), lambda i: (0, i, 0)),
        out_shape=jax.ShapeDtypeStruct((16, _N, _CD), jnp.float32),
    )(x_cm, w_cm, b_row)


def _combine_body(agg_ref, s_ref, w_ref, h_ref, sc_ref):
    nrm = jnp.sqrt(jnp.sum(w_ref[...] * w_ref[...])) + 1e-16
    acc = jnp.zeros((agg_ref.shape[1], 1), jnp.float32)
    for c in range(_NCHUNK):
        h = jnp.maximum(agg_ref[c] + s_ref[c], 0.0)
        h_ref[c] = h
        acc = acc + jnp.dot(h, w_ref[c][:, None],
                            preferred_element_type=jnp.float32)
    sc_ref[...] = jnp.tanh(acc / nrm)


def _combine_call(agg_cm, s_cm, w_ck):
    nt = _N // _RC
    return pl.pallas_call(
        _combine_body,
        grid=(nt,),
        in_specs=[
            pl.BlockSpec((_NCHUNK, _RC, _CD), lambda i: (0, i, 0)),
            pl.BlockSpec((_NCHUNK, _RC, _CD), lambda i: (0, i, 0)),
            pl.BlockSpec((_NCHUNK, _CD), lambda i: (0, 0)),
        ],
        out_specs=[
            pl.BlockSpec((_NCHUNK, _RC, _CD), lambda i: (0, i, 0)),
            pl.BlockSpec((_RC, 1), lambda i: (i, 0)),
        ],
        out_shape=[
            jax.ShapeDtypeStruct((_NCHUNK, _N, _CD), jnp.float32),
            jax.ShapeDtypeStruct((_N, 1), jnp.float32),
        ],
    )(agg_cm, s_cm, w_ck)


def _sortable(x):
    bits = lax.bitcast_convert_type(x, jnp.int32)
    return jnp.where(bits >= 0, bits, bits ^ jnp.int32(0x7FFFFFFF))


def _topk_body(kk, sc_ref, nm_ref, o_ref):
    masked = jnp.where(nm_ref[...] > 0, sc_ref[...], -1e9)
    key = _sortable(masked)

    def cnt_ge(t):
        return jnp.sum((key >= t).astype(jnp.int32))

    t = jnp.int32(-2147483648)
    for bit in range(31, -1, -1):
        step = jnp.int32(-2147483648) if bit == 31 else jnp.int32(1 << bit)
        cand = t + step
        t = jnp.where(cnt_ge(cand) >= kk, cand, t)

    need = kk - cnt_ge(t + 1)
    idx = (lax.broadcasted_iota(jnp.int32, (_SROW, 128), 0) * 128
           + lax.broadcasted_iota(jnp.int32, (_SROW, 128), 1))
    is_t = key == t
    c = jnp.int32(0)
    for bit in range(13, -1, -1):
        cand = c + jnp.int32(1 << bit)
        tie_cnt = jnp.sum((is_t & (idx < cand)).astype(jnp.int32))
        c = jnp.where(tie_cnt < need, cand, c)
    cstar = c + 1

    lane = lax.broadcasted_iota(jnp.int32, (8, 128), 1)
    o_ref[...] = jnp.where(lane == 1, cstar, t)


def _topk_call(kk, sc2d, nm2d):
    return pl.pallas_call(
        functools.partial(_topk_body, kk),
        grid=(1,),
        in_specs=[
            pl.BlockSpec((_SROW, 128), lambda i: (0, 0)),
            pl.BlockSpec((_SROW, 128), lambda i: (0, 0)),
        ],
        out_specs=pl.BlockSpec((8, 128), lambda i: (0, 0)),
        out_shape=jax.ShapeDtypeStruct((8, 128), jnp.int32),
    )(sc2d, nm2d)


def _gate_body(kk, h_ref, sc_ref, nm_ref, t_ref, hg_ref, nmo_ref, ro_ref):
    i = pl.program_id(0)
    t = t_ref[0, 0]
    cstar = t_ref[0, 1]
    masked = jnp.where(nm_ref[...] > 0, sc_ref[...], -1e9)
    key = _sortable(masked)
    idx = i * _RC + lax.broadcasted_iota(jnp.int32, (_RC, 1), 0)
    sel = (key > t) | ((key == t) & (idx < cstar))
    keep = sel.astype(jnp.float32)
    nmo_ref[...] = keep
    gate = sc_ref[...] * keep

    @pl.when(i == 0)
    def _():
        ro_ref[...] = jnp.zeros((_NCHUNK, _CD), jnp.float32)

    parts = []
    for c in range(_NCHUNK):
        hg = h_ref[c] * gate
        hg_ref[c] = hg
        parts.append(jnp.sum(hg, axis=0, keepdims=True) * (1.0 / kk))
    ro_ref[...] = ro_ref[...] + jnp.concatenate(parts, axis=0)


def _gate_call(kk, h_cm, sc_col, nm_col, taux):
    nt = _N // _RC
    return pl.pallas_call(
        functools.partial(_gate_body, kk),
        grid=(nt,),
        in_specs=[
            pl.BlockSpec((_NCHUNK, _RC, _CD), lambda i: (0, i, 0)),
            pl.BlockSpec((_RC, 1), lambda i: (i, 0)),
            pl.BlockSpec((_RC, 1), lambda i: (i, 0)),
            pl.BlockSpec((8, 128), lambda i: (0, 0)),
        ],
        out_specs=[
            pl.BlockSpec((_NCHUNK, _RC, _CD), lambda i: (0, i, 0)),
            pl.BlockSpec((_RC, 1), lambda i: (i, 0)),
            pl.BlockSpec((_NCHUNK, _CD), lambda i: (0, 0)),
        ],
        out_shape=[
            jax.ShapeDtypeStruct((_NCHUNK, _N, _CD), jnp.float32),
            jax.ShapeDtypeStruct((_N, 1), jnp.float32),
            jax.ShapeDtypeStruct((_NCHUNK, _CD), jnp.float32),
        ],
    )(h_cm, sc_col, nm_col, taux)


def _mlp_body(r1_ref, r2_ref, r3_ref, w1_ref, b1_ref, w2_ref, b2_ref,
              w3_ref, b3_ref, o_ref):
    z = r1_ref[...] + r2_ref[...] + r3_ref[...]
    z = jnp.maximum(jnp.dot(z, w1_ref[...],
                            preferred_element_type=jnp.float32)
                    + b1_ref[...], 0.0)
    z = jnp.maximum(jnp.dot(z, w2_ref[...],
                            preferred_element_type=jnp.float32)
                    + b2_ref[...], 0.0)
    o_ref[...] = jnp.dot(z, w3_ref[...],
                         preferred_element_type=jnp.float32) + b3_ref[...]


def _mlp_call(r1, r2, r3, w1, b1, w2, b2, w3, b3):
    no = w3.shape[1]
    specs = [pl.BlockSpec(a.shape, lambda i: tuple(0 for _ in a.shape))
             for a in (r1, r2, r3, w1, b1, w2, b2, w3, b3)]
    return pl.pallas_call(
        _mlp_body,
        grid=(1,),
        in_specs=specs,
        out_specs=pl.BlockSpec((1, no), lambda i: (0, 0)),
        out_shape=jax.ShapeDtypeStruct((1, no), jnp.float32),
    )(r1, r2, r3, w1, b1, w2, b2, w3, b3)


def _pad_img(col, fill):
    return jnp.pad(col[:, 0], (0, _NPAD - _N),
                   constant_values=fill).reshape(_SROW, 128)


def kernel(x, edge_index, batch, params):
    src = edge_index[0].astype(jnp.int32)
    dst = edge_index[1].astype(jnp.int32)
    pk = (dst * 16384 + src).astype(jnp.int32)
    p = params

    def layer(h_cm, cp, w_pool, kk, nm_col):
        ic = h_cm.shape[0]
        w_cm = jnp.concatenate([cp['Wk'], cp['Wq'], cp['Wv'], cp['Ws']],
                               axis=1).reshape(ic, _CD, 2048)
        b_row = jnp.concatenate([cp['bk'], cp['bq'], cp['bv'], cp['bs']])[None]
        y = _proj_call(h_cm, w_cm, b_row)
        k = y[0:4].reshape(_NCHUNK * _N, _CD)
        q = y[4:8].reshape(_NCHUNK * _N, _CD)
        v = y[8:12].reshape(_NCHUNK * _N, _CD)
        s = y[12:16]
        agg = _edge_call(pk, k, q, v)[0].reshape(_NCHUNK, _N, _CD)
        h4, score = _combine_call(agg, s, w_pool.reshape(_NCHUNK, _CD))
        taux = _topk_call(kk, _pad_img(score, 0.0), _pad_img(nm_col, 0.0))
        hg, nm_new, ro = _gate_call(kk, h4, score, nm_col, taux)
        return hg, nm_new, ro

    x_cm = x.reshape(_N, 2, _CD).transpose(1, 0, 2)
    nmask = jnp.ones((_N, 1), jnp.float32)
    h, nmask, r1 = layer(x_cm, p['c1'], p['p1'], _K1, nmask)
    h, nmask, r2 = layer(h, p['c2'], p['p2'], _K2, nmask)
    h, nmask, r3 = layer(h, p['c3'], p['p3'], _K3, nmask)

    return _mlp_call(r1.reshape(1, _D), r2.reshape(1, _D), r3.reshape(1, _D),
                     p['l1W'], p['l1b'][None], p['l2W'], p['l2b'][None],
                     p['l3W'], p['l3b'][None])
